# baseline (speedup 1.0000x reference)
import dataclasses
import functools

import jax
import jax.numpy as jnp
from jax import lax
from jax.experimental import pallas as pl
from jax.experimental.pallas import tpu as pltpu
from jax.experimental.pallas import tpu_sc as plsc

_N = 16
_M = 256
_B = 16384
_L = 200
_NW = 32
_ROWS_W = _B // _NW
_CROWS = 64
_NCH = _ROWS_W // _CROWS
_LANES = 16
_COLS = tuple(range(0, _L - _LANES + 1, _LANES)) + (_L - _LANES,)

_cp = pltpu.CompilerParams()
if "needs_layout_passes" in pltpu.CompilerParams.__dataclass_fields__:
    _cp = dataclasses.replace(_cp, needs_layout_passes=False)
if "use_tc_tiling_on_sc" in pltpu.CompilerParams.__dataclass_fields__:
    _cp = dataclasses.replace(_cp, use_tc_tiling_on_sc=True)


@functools.partial(
    pl.kernel,
    mesh=plsc.VectorSubcoreMesh(core_axis_name="c", subcore_axis_name="s"),
    compiler_params=_cp,
    out_type=(
        jax.ShapeDtypeStruct((_B, _L), jnp.float32),
        jax.ShapeDtypeStruct((_B, _L), jnp.float32),
    ),
    scratch_types=[
        pltpu.VMEM((_N,), jnp.float32),
        pltpu.VMEM((_M,), jnp.float32),
        pltpu.VMEM((_CROWS, _L), jnp.int32),
        pltpu.VMEM((_CROWS, _L), jnp.int32),
        pltpu.VMEM((_CROWS, _L), jnp.float32),
        pltpu.VMEM((_CROWS, _L), jnp.float32),
        pltpu.SemaphoreType.DMA,
        pltpu.SemaphoreType.DMA,
        pltpu.SemaphoreType.DMA,
        pltpu.SemaphoreType.DMA,
    ],
)
def _lookup_sc(theta_hbm, mag_hbm, cos_hbm, exp_hbm, phase_hbm, magv_hbm,
               cos_v, exp_v, idx0_v, idx1_v, val0_v, val1_v,
               sin0, sin1, sout0, sout1):
    cid = lax.axis_index("c")
    sid = lax.axis_index("s")
    wid = sid * 2 + cid
    base = wid * _ROWS_W

    pltpu.sync_copy(cos_hbm, cos_v)
    pltpu.sync_copy(exp_hbm, exp_v)

    idx_bufs = (idx0_v, idx1_v)
    val_bufs = (val0_v, val1_v)
    sins = (sin0, sin1)
    souts = (sout0, sout1)

    def do_array(src_hbm, dst_hbm, table_v, mask):
        for b in range(2):
            pltpu.async_copy(
                src_hbm.at[pl.ds(base + b * _CROWS, _CROWS)], idx_bufs[b],
                sins[b])
        for ch in range(_NCH):
            b = ch % 2
            idx_v, val_v = idx_bufs[b], val_bufs[b]
            row = base + ch * _CROWS
            pltpu.make_async_copy(
                src_hbm.at[pl.ds(base, _CROWS)], idx_v, sins[b]).wait()
            if ch >= 2:
                pltpu.make_async_copy(
                    val_v, dst_hbm.at[pl.ds(base, _CROWS)], souts[b]).wait()

            @plsc.parallel_loop(0, _CROWS, unroll=2)
            def _rows(r):
                for j in _COLS:
                    sl = (r, pl.ds(j, _LANES))
                    idx = jnp.bitwise_and(idx_v[sl], mask)
                    val_v[sl] = plsc.load_gather(table_v, [idx])

            pltpu.async_copy(
                val_v, dst_hbm.at[pl.ds(row, _CROWS)], souts[b])
            if ch + 2 < _NCH:
                pltpu.async_copy(
                    src_hbm.at[pl.ds(base + (ch + 2) * _CROWS, _CROWS)],
                    idx_v, sins[b])
        for b in range(min(2, _NCH)):
            pltpu.make_async_copy(
                val_bufs[b], dst_hbm.at[pl.ds(base, _CROWS)], souts[b]).wait()

    do_array(theta_hbm, phase_hbm, cos_v, _N - 1)
    do_array(mag_hbm, magv_hbm, exp_v, _M - 1)


def kernel(theta_indices, mag_indices, cos_table, exp_table):
    phase, mag = _lookup_sc(theta_indices, mag_indices, cos_table, exp_table)
    return (phase, mag)


# speedup vs baseline: 1.9633x; 1.9633x over previous
import dataclasses
import functools

import jax
import jax.numpy as jnp
from jax import lax
from jax.experimental import pallas as pl
from jax.experimental.pallas import tpu as pltpu
from jax.experimental.pallas import tpu_sc as plsc

_N = 16
_M = 256
_B = 16384
_L = 200
_NT = _B * _L
_NW = 32
_PW = _NT // _NW
_CHUNK = 12_800
_NCH = _PW // _CHUNK
_LANES = 16

_cp = pltpu.CompilerParams()
if "needs_layout_passes" in pltpu.CompilerParams.__dataclass_fields__:
    _cp = dataclasses.replace(_cp, needs_layout_passes=False)


@functools.partial(
    pl.kernel,
    mesh=plsc.VectorSubcoreMesh(core_axis_name="c", subcore_axis_name="s"),
    compiler_params=_cp,
    out_type=(
        jax.ShapeDtypeStruct((_NT,), jnp.float32),
        jax.ShapeDtypeStruct((_NT,), jnp.float32),
    ),
    scratch_types=[
        pltpu.VMEM((_N,), jnp.float32),
        pltpu.VMEM((_M,), jnp.float32),
        pltpu.VMEM((_CHUNK,), jnp.int32),
        pltpu.VMEM((_CHUNK,), jnp.int32),
        pltpu.VMEM((_CHUNK,), jnp.float32),
        pltpu.VMEM((_CHUNK,), jnp.float32),
        pltpu.SemaphoreType.DMA,
        pltpu.SemaphoreType.DMA,
        pltpu.SemaphoreType.DMA,
        pltpu.SemaphoreType.DMA,
    ],
)
def _lookup_sc(theta_hbm, mag_hbm, cos_hbm, exp_hbm, phase_hbm, magv_hbm,
               cos_v, exp_v, idx0_v, idx1_v, val0_v, val1_v,
               sin0, sin1, sout0, sout1):
    cid = lax.axis_index("c")
    sid = lax.axis_index("s")
    wid = sid * 2 + cid
    base = wid * _PW

    pltpu.sync_copy(cos_hbm, cos_v)
    pltpu.sync_copy(exp_hbm, exp_v)

    idx_bufs = (idx0_v, idx1_v)
    val_bufs = (val0_v, val1_v)
    sins = (sin0, sin1)
    souts = (sout0, sout1)

    def do_array(src_hbm, dst_hbm, table_v, mask):
        for b in range(2):
            pltpu.async_copy(
                src_hbm.at[pl.ds(base + b * _CHUNK, _CHUNK)], idx_bufs[b],
                sins[b])
        for ch in range(_NCH):
            b = ch % 2
            idx_v, val_v = idx_bufs[b], val_bufs[b]
            pltpu.make_async_copy(
                src_hbm.at[pl.ds(base, _CHUNK)], idx_v, sins[b]).wait()
            if ch >= 2:
                pltpu.make_async_copy(
                    val_v, dst_hbm.at[pl.ds(base, _CHUNK)], souts[b]).wait()

            @plsc.parallel_loop(0, _CHUNK, step=_LANES * 8, unroll=2)
            def _vecs(i):
                for u in range(8):
                    sl = pl.ds(i + u * _LANES, _LANES)
                    idx = jnp.bitwise_and(idx_v[sl], mask)
                    val_v[sl] = plsc.load_gather(table_v, [idx])

            pltpu.async_copy(
                val_v, dst_hbm.at[pl.ds(base + ch * _CHUNK, _CHUNK)],
                souts[b])
            if ch + 2 < _NCH:
                pltpu.async_copy(
                    src_hbm.at[pl.ds(base + (ch + 2) * _CHUNK, _CHUNK)],
                    idx_v, sins[b])
        for b in range(min(2, _NCH)):
            pltpu.make_async_copy(
                val_bufs[b], dst_hbm.at[pl.ds(base, _CHUNK)], souts[b]).wait()

    do_array(theta_hbm, phase_hbm, cos_v, _N - 1)
    do_array(mag_hbm, magv_hbm, exp_v, _M - 1)


def _to_tiled_flat(x):
    return (x.reshape(128, 128, 25, 8)
             .transpose(2, 0, 3, 1)
             .reshape(_NT))


def _from_tiled_flat(y):
    return (y.reshape(25, 128, 8, 128)
             .transpose(1, 3, 0, 2)
             .reshape(_B, _L))


def kernel(theta_indices, mag_indices, cos_table, exp_table):
    th = _to_tiled_flat(theta_indices)
    mg = _to_tiled_flat(mag_indices)
    phase, mag = _lookup_sc(th, mg, cos_table, exp_table)
    return (_from_tiled_flat(phase), _from_tiled_flat(mag))


# speedup vs baseline: 2.0967x; 1.0679x over previous
import dataclasses
import functools

import jax
import jax.numpy as jnp
from jax import lax
from jax.experimental import pallas as pl
from jax.experimental.pallas import tpu as pltpu
from jax.experimental.pallas import tpu_sc as plsc

_N = 16
_M = 256
_B = 16384
_L = 200
_NT = _B * _L
_NW = 32
_PW = _NT // _NW
_CHUNK = 25_600
_NCH = _PW // _CHUNK
_LANES = 16

_cp = pltpu.CompilerParams()
if "needs_layout_passes" in pltpu.CompilerParams.__dataclass_fields__:
    _cp = dataclasses.replace(_cp, needs_layout_passes=False)


@functools.partial(
    pl.kernel,
    mesh=plsc.VectorSubcoreMesh(core_axis_name="c", subcore_axis_name="s"),
    compiler_params=_cp,
    out_type=(
        jax.ShapeDtypeStruct((_NT,), jnp.float32),
        jax.ShapeDtypeStruct((_NT,), jnp.float32),
    ),
    scratch_types=[
        pltpu.VMEM((_N,), jnp.float32),
        pltpu.VMEM((_M,), jnp.float32),
        pltpu.VMEM((_CHUNK,), jnp.int32),
        pltpu.VMEM((_CHUNK,), jnp.int32),
        pltpu.VMEM((_CHUNK,), jnp.float32),
        pltpu.VMEM((_CHUNK,), jnp.float32),
        pltpu.SemaphoreType.DMA,
        pltpu.SemaphoreType.DMA,
        pltpu.SemaphoreType.DMA,
        pltpu.SemaphoreType.DMA,
    ],
)
def _lookup_sc(theta_hbm, mag_hbm, cos_hbm, exp_hbm, phase_hbm, magv_hbm,
               cos_v, exp_v, idx0_v, idx1_v, val0_v, val1_v,
               sin0, sin1, sout0, sout1):
    cid = lax.axis_index("c")
    sid = lax.axis_index("s")
    wid = sid * 2 + cid
    base = wid * _PW

    pltpu.sync_copy(cos_hbm, cos_v)
    pltpu.sync_copy(exp_hbm, exp_v)

    idx_bufs = (idx0_v, idx1_v)
    val_bufs = (val0_v, val1_v)
    sins = (sin0, sin1)
    souts = (sout0, sout1)

    def do_array(src_hbm, dst_hbm, table_v, mask):
        for b in range(2):
            pltpu.async_copy(
                src_hbm.at[pl.ds(base + b * _CHUNK, _CHUNK)], idx_bufs[b],
                sins[b])
        for ch in range(_NCH):
            b = ch % 2
            idx_v, val_v = idx_bufs[b], val_bufs[b]
            pltpu.make_async_copy(
                src_hbm.at[pl.ds(base, _CHUNK)], idx_v, sins[b]).wait()
            if ch >= 2:
                pltpu.make_async_copy(
                    val_v, dst_hbm.at[pl.ds(base, _CHUNK)], souts[b]).wait()

            @plsc.parallel_loop(0, _CHUNK, step=_LANES * 8, unroll=2)
            def _vecs(i):
                for u in range(8):
                    sl = pl.ds(i + u * _LANES, _LANES)
                    idx = jnp.bitwise_and(idx_v[sl], mask)
                    val_v[sl] = plsc.load_gather(table_v, [idx])

            pltpu.async_copy(
                val_v, dst_hbm.at[pl.ds(base + ch * _CHUNK, _CHUNK)],
                souts[b])
            if ch + 2 < _NCH:
                pltpu.async_copy(
                    src_hbm.at[pl.ds(base + (ch + 2) * _CHUNK, _CHUNK)],
                    idx_v, sins[b])
        for b in range(min(2, _NCH)):
            pltpu.make_async_copy(
                val_bufs[b], dst_hbm.at[pl.ds(base, _CHUNK)], souts[b]).wait()

    do_array(theta_hbm, phase_hbm, cos_v, _N - 1)
    do_array(mag_hbm, magv_hbm, exp_v, _M - 1)


def _to_tiled_flat(x):
    return (x.reshape(128, 128, 25, 8)
             .transpose(2, 0, 3, 1)
             .reshape(_NT))


def _from_tiled_flat(y):
    return (y.reshape(25, 128, 8, 128)
             .transpose(1, 3, 0, 2)
             .reshape(_B, _L))


def kernel(theta_indices, mag_indices, cos_table, exp_table):
    th = _to_tiled_flat(theta_indices)
    mg = _to_tiled_flat(mag_indices)
    phase, mag = _lookup_sc(th, mg, cos_table, exp_table)
    return (_from_tiled_flat(phase), _from_tiled_flat(mag))


# speedup vs baseline: 2.2442x; 1.0703x over previous
import dataclasses
import functools

import jax
import jax.numpy as jnp
from jax import lax
from jax.experimental import pallas as pl
from jax.experimental.pallas import tpu as pltpu
from jax.experimental.pallas import tpu_sc as plsc

_N = 16
_M = 256
_B = 16384
_L = 200
_NT = _B * _L
_NW = 32
_PW = _NT // _NW
_CHUNK = 25_600
_NCH = _PW // _CHUNK
_LANES = 16

_cp = pltpu.CompilerParams()
if "needs_layout_passes" in pltpu.CompilerParams.__dataclass_fields__:
    _cp = dataclasses.replace(_cp, needs_layout_passes=False)


@functools.partial(
    pl.kernel,
    mesh=plsc.VectorSubcoreMesh(core_axis_name="c", subcore_axis_name="s"),
    compiler_params=_cp,
    out_type=(
        jax.ShapeDtypeStruct((_NT,), jnp.float32),
        jax.ShapeDtypeStruct((_NT,), jnp.float32),
    ),
    scratch_types=[
        pltpu.VMEM((_N,), jnp.float32),
        pltpu.VMEM((_M,), jnp.float32),
        pltpu.VMEM((_CHUNK,), jnp.int32),
        pltpu.VMEM((_CHUNK,), jnp.int32),
        pltpu.VMEM((_CHUNK,), jnp.float32),
        pltpu.VMEM((_CHUNK,), jnp.float32),
        pltpu.SemaphoreType.DMA,
        pltpu.SemaphoreType.DMA,
        pltpu.SemaphoreType.DMA,
        pltpu.SemaphoreType.DMA,
    ],
)
def _lookup_sc(theta_hbm, mag_hbm, cos_hbm, exp_hbm, phase_hbm, magv_hbm,
               cos_v, exp_v, idx0_v, idx1_v, val0_v, val1_v,
               sin0, sin1, sout0, sout1):
    cid = lax.axis_index("c")
    sid = lax.axis_index("s")
    wid = sid * 2 + cid
    base = wid * _PW

    pltpu.sync_copy(cos_hbm, cos_v)
    pltpu.sync_copy(exp_hbm, exp_v)

    idx_bufs = (idx0_v, idx1_v)
    val_bufs = (val0_v, val1_v)
    sins = (sin0, sin1)
    souts = (sout0, sout1)

    def do_array(src_hbm, dst_hbm, table_v, mask, table_reg=None):
        for b in range(2):
            pltpu.async_copy(
                src_hbm.at[pl.ds(base + b * _CHUNK, _CHUNK)], idx_bufs[b],
                sins[b])
        for ch in range(_NCH):
            b = ch % 2
            idx_v, val_v = idx_bufs[b], val_bufs[b]
            pltpu.make_async_copy(
                src_hbm.at[pl.ds(base, _CHUNK)], idx_v, sins[b]).wait()
            if ch >= 2:
                pltpu.make_async_copy(
                    val_v, dst_hbm.at[pl.ds(base, _CHUNK)], souts[b]).wait()

            @plsc.parallel_loop(0, _CHUNK, step=_LANES * 8, unroll=2)
            def _vecs(i):
                for u in range(8):
                    sl = pl.ds(i + u * _LANES, _LANES)
                    idx = jnp.bitwise_and(idx_v[sl], mask)
                    if table_reg is not None:
                        val_v[sl] = lax.gather(
                            table_reg, idx[:, None],
                            lax.GatherDimensionNumbers(
                                offset_dims=(),
                                collapsed_slice_dims=(0,),
                                start_index_map=(0,)),
                            slice_sizes=(1,),
                            mode=lax.GatherScatterMode.PROMISE_IN_BOUNDS)
                    else:
                        val_v[sl] = plsc.load_gather(table_v, [idx])

            pltpu.async_copy(
                val_v, dst_hbm.at[pl.ds(base + ch * _CHUNK, _CHUNK)],
                souts[b])
            if ch + 2 < _NCH:
                pltpu.async_copy(
                    src_hbm.at[pl.ds(base + (ch + 2) * _CHUNK, _CHUNK)],
                    idx_v, sins[b])
        for b in range(min(2, _NCH)):
            pltpu.make_async_copy(
                val_bufs[b], dst_hbm.at[pl.ds(base, _CHUNK)], souts[b]).wait()

    cos_reg = cos_v[pl.ds(0, _LANES)]
    do_array(theta_hbm, phase_hbm, cos_v, _N - 1, table_reg=cos_reg)
    do_array(mag_hbm, magv_hbm, exp_v, _M - 1)


def _to_tiled_flat(x):
    return (x.reshape(128, 128, 25, 8)
             .transpose(2, 0, 3, 1)
             .reshape(_NT))


def _from_tiled_flat(y):
    return (y.reshape(25, 128, 8, 128)
             .transpose(1, 3, 0, 2)
             .reshape(_B, _L))


def kernel(theta_indices, mag_indices, cos_table, exp_table):
    th = _to_tiled_flat(theta_indices)
    mg = _to_tiled_flat(mag_indices)
    phase, mag = _lookup_sc(th, mg, cos_table, exp_table)
    return (_from_tiled_flat(phase), _from_tiled_flat(mag))
